# baseline (device time: 34721 ns/iter reference)
import jax
import jax.numpy as jnp
from jax import lax
from jax.experimental import pallas as pl
from jax.experimental.pallas import tpu as pltpu

GRID = 8


def kernel(x):
    m, n = x.shape
    mc = m // GRID

    def body(x_ref, out_ref, acc_ref, send_sem, recv_sem):
        i = pl.program_id(0)

        acc_ref[0, pl.ds(i * mc, mc), :] = jnp.sum(
            x_ref[:, :], axis=1, keepdims=True
        )

        @pl.when(i == GRID - 1)
        def _():
            my_x = lax.axis_index("x")
            my_y = lax.axis_index("y")
            peer = (my_x, 1 - my_y)

            barrier = pltpu.get_barrier_semaphore()
            pl.semaphore_signal(
                barrier,
                inc=1,
                device_id=peer,
                device_id_type=pl.DeviceIdType.MESH,
            )
            pl.semaphore_wait(barrier, 1)

            rdma = pltpu.make_async_remote_copy(
                src_ref=acc_ref.at[0],
                dst_ref=acc_ref.at[1],
                send_sem=send_sem,
                recv_sem=recv_sem,
                device_id=peer,
                device_id_type=pl.DeviceIdType.MESH,
            )
            rdma.start()
            rdma.wait()

            out_ref[:, :] = (acc_ref[0, :, :] + acc_ref[1, :, :]) * (
                1.0 / (2 * n)
            )

    return pl.pallas_call(
        body,
        grid=(GRID,),
        out_shape=jax.ShapeDtypeStruct((m, 1), jnp.float32),
        in_specs=[
            pl.BlockSpec((mc, n), lambda i: (i, 0), memory_space=pltpu.VMEM)
        ],
        out_specs=pl.BlockSpec((m, 1), lambda i: (0, 0), memory_space=pltpu.VMEM),
        scratch_shapes=[
            pltpu.VMEM((2, m, 1), jnp.float32),
            pltpu.SemaphoreType.DMA,
            pltpu.SemaphoreType.DMA,
        ],
        compiler_params=pltpu.CompilerParams(
            collective_id=0, dimension_semantics=("arbitrary",)
        ),
    )(x)


# device time: 34658 ns/iter; 1.0018x vs baseline; 1.0018x over previous
import jax
import jax.numpy as jnp
from jax import lax
from jax.experimental import pallas as pl
from jax.experimental.pallas import tpu as pltpu

GRID = 8


def kernel(x):
    m, n = x.shape
    mc = m // GRID

    def body(x_ref, out_ref, acc_ref, send_sem, recv_sem):
        i = pl.program_id(0)

        parts = [x_ref[:, pl.ds(k * 128, 128)] for k in range(n // 128)]
        while len(parts) > 1:
            parts = [
                parts[j] + parts[j + 1] for j in range(0, len(parts), 2)
            ]
        acc_ref[0, pl.ds(i * mc, mc), :] = jnp.sum(
            parts[0], axis=1, keepdims=True
        )

        @pl.when(i == GRID - 1)
        def _():
            my_x = lax.axis_index("x")
            my_y = lax.axis_index("y")
            peer = (my_x, 1 - my_y)

            barrier = pltpu.get_barrier_semaphore()
            pl.semaphore_signal(
                barrier,
                inc=1,
                device_id=peer,
                device_id_type=pl.DeviceIdType.MESH,
            )
            pl.semaphore_wait(barrier, 1)

            rdma = pltpu.make_async_remote_copy(
                src_ref=acc_ref.at[0],
                dst_ref=acc_ref.at[1],
                send_sem=send_sem,
                recv_sem=recv_sem,
                device_id=peer,
                device_id_type=pl.DeviceIdType.MESH,
            )
            rdma.start()
            rdma.wait()

            out_ref[:, :] = (acc_ref[0, :, :] + acc_ref[1, :, :]) * (
                1.0 / (2 * n)
            )

    return pl.pallas_call(
        body,
        grid=(GRID,),
        out_shape=jax.ShapeDtypeStruct((m, 1), jnp.float32),
        in_specs=[
            pl.BlockSpec((mc, n), lambda i: (i, 0), memory_space=pltpu.VMEM)
        ],
        out_specs=pl.BlockSpec((m, 1), lambda i: (0, 0), memory_space=pltpu.VMEM),
        scratch_shapes=[
            pltpu.VMEM((2, m, 1), jnp.float32),
            pltpu.SemaphoreType.DMA,
            pltpu.SemaphoreType.DMA,
        ],
        compiler_params=pltpu.CompilerParams(
            collective_id=0, dimension_semantics=("arbitrary",)
        ),
    )(x)


# device time: 13208 ns/iter; 2.6288x vs baseline; 2.6240x over previous
import jax
import jax.numpy as jnp
from jax import lax
from jax.experimental import pallas as pl
from jax.experimental.pallas import tpu as pltpu

GRID = 8


def kernel(x):
    m, n = x.shape
    mc = m // GRID
    r = mc // 128

    def body(x_ref, out_ref, cacc_ref, send_sem, recv_sem):
        i = pl.program_id(0)

        my_x = lax.axis_index("x")
        my_y = lax.axis_index("y")
        peer = (my_x, 1 - my_y)

        @pl.when(i == 0)
        def _():
            barrier = pltpu.get_barrier_semaphore()
            pl.semaphore_signal(
                barrier,
                inc=1,
                device_id=peer,
                device_id_type=pl.DeviceIdType.MESH,
            )
            pl.semaphore_wait(barrier, 1)

        ones = jnp.ones((1, n), jnp.float32)
        rs = lax.dot_general(
            ones,
            x_ref[:, :],
            dimension_numbers=(((1,), (1,)), ((), ())),
            preferred_element_type=jnp.float32,
        )
        for g in range(r):
            cacc_ref[0, pl.ds(i * r + g, 1), :] = rs[
                :, g * 128 : (g + 1) * 128
            ]

        @pl.when(i == GRID - 1)
        def _():
            rdma = pltpu.make_async_remote_copy(
                src_ref=cacc_ref.at[0],
                dst_ref=cacc_ref.at[1],
                send_sem=send_sem,
                recv_sem=recv_sem,
                device_id=peer,
                device_id_type=pl.DeviceIdType.MESH,
            )
            rdma.start()
            rdma.wait()

            out_ref[:, :] = (cacc_ref[0] + cacc_ref[1]) * (1.0 / (2 * n))

    out = pl.pallas_call(
        body,
        grid=(GRID,),
        out_shape=jax.ShapeDtypeStruct((m // 128, 128), jnp.float32),
        in_specs=[
            pl.BlockSpec((mc, n), lambda i: (i, 0), memory_space=pltpu.VMEM)
        ],
        out_specs=pl.BlockSpec(
            (m // 128, 128), lambda i: (0, 0), memory_space=pltpu.VMEM
        ),
        scratch_shapes=[
            pltpu.VMEM((2, m // 128, 128), jnp.float32),
            pltpu.SemaphoreType.DMA,
            pltpu.SemaphoreType.DMA,
        ],
        compiler_params=pltpu.CompilerParams(
            collective_id=0, dimension_semantics=("arbitrary",)
        ),
    )(x)
    return out.reshape(m, 1)


# device time: 13200 ns/iter; 2.6304x vs baseline; 1.0006x over previous
import jax
import jax.numpy as jnp
from jax import lax
from jax.experimental import pallas as pl
from jax.experimental.pallas import tpu as pltpu

GRID = 8


def kernel(x):
    m, n = x.shape
    mc = m // GRID
    r = mc // 128

    def body(x_ref, out_ref, cacc_ref, send_sems, recv_sems):
        i = pl.program_id(0)

        my_x = lax.axis_index("x")
        my_y = lax.axis_index("y")
        peer = (my_x, 1 - my_y)

        @pl.when(i == 0)
        def _():
            barrier = pltpu.get_barrier_semaphore()
            pl.semaphore_signal(
                barrier,
                inc=1,
                device_id=peer,
                device_id_type=pl.DeviceIdType.MESH,
            )
            pl.semaphore_wait(barrier, 1)

        ones = jnp.ones((1, n), jnp.float32)
        rs = lax.dot_general(
            ones,
            x_ref[:, :],
            dimension_numbers=(((1,), (1,)), ((), ())),
            preferred_element_type=jnp.float32,
        )
        for g in range(r):
            cacc_ref[0, pl.ds(i * r + g, 1), :] = rs[
                :, g * 128 : (g + 1) * 128
            ]

        half = (m // 128) // 2

        def make_rdma(h):
            lo = h * half
            return pltpu.make_async_remote_copy(
                src_ref=cacc_ref.at[0, pl.ds(lo, half)],
                dst_ref=cacc_ref.at[1, pl.ds(lo, half)],
                send_sem=send_sems.at[h],
                recv_sem=recv_sems.at[h],
                device_id=peer,
                device_id_type=pl.DeviceIdType.MESH,
            )

        @pl.when(i == GRID // 2 - 1)
        def _():
            make_rdma(0).start()

        @pl.when(i == GRID - 1)
        def _():
            rdma1 = make_rdma(1)
            rdma1.start()
            rdma0 = make_rdma(0)
            rdma0.wait_send()
            rdma0.wait_recv()
            rdma1.wait_send()
            rdma1.wait_recv()

            out_ref[:, :] = (cacc_ref[0] + cacc_ref[1]) * (1.0 / (2 * n))

    out = pl.pallas_call(
        body,
        grid=(GRID,),
        out_shape=jax.ShapeDtypeStruct((m // 128, 128), jnp.float32),
        in_specs=[
            pl.BlockSpec((mc, n), lambda i: (i, 0), memory_space=pltpu.VMEM)
        ],
        out_specs=pl.BlockSpec(
            (m // 128, 128), lambda i: (0, 0), memory_space=pltpu.VMEM
        ),
        scratch_shapes=[
            pltpu.VMEM((2, m // 128, 128), jnp.float32),
            pltpu.SemaphoreType.DMA((2,)),
            pltpu.SemaphoreType.DMA((2,)),
        ],
        compiler_params=pltpu.CompilerParams(
            collective_id=0, dimension_semantics=("arbitrary",)
        ),
    )(x)
    return out.reshape(m, 1)


# device time: 11744 ns/iter; 2.9565x vs baseline; 1.1240x over previous
import jax
import jax.numpy as jnp
from jax import lax
from jax.experimental import pallas as pl
from jax.experimental.pallas import tpu as pltpu

GRID = 8


def kernel(x):
    m, n = x.shape
    mc = m // GRID
    r = mc // 128

    def body(x_ref, out_ref, cacc_ref, send_sems, recv_sems):
        i = pl.program_id(0)

        my_x = lax.axis_index("x")
        my_y = lax.axis_index("y")
        peer = (my_x, 1 - my_y)
        barrier = pltpu.get_barrier_semaphore()

        @pl.when(i == 0)
        def _():
            pl.semaphore_signal(
                barrier,
                inc=1,
                device_id=peer,
                device_id_type=pl.DeviceIdType.MESH,
            )

        parts = [x_ref[:, pl.ds(k * 128, 128)] for k in range(n // 128)]
        while len(parts) > 1:
            parts = [parts[j] + parts[j + 1] for j in range(0, len(parts), 2)]
        ones = jnp.ones((1, 128), jnp.float32)
        rs = lax.dot_general(
            ones,
            parts[0],
            dimension_numbers=(((1,), (1,)), ((), ())),
            preferred_element_type=jnp.float32,
        )
        for g in range(r):
            cacc_ref[0, pl.ds(i * r + g, 1), :] = rs[
                :, g * 128 : (g + 1) * 128
            ]

        half = (m // 128) // 2

        def make_rdma(h):
            lo = h * half
            return pltpu.make_async_remote_copy(
                src_ref=cacc_ref.at[0, pl.ds(lo, half)],
                dst_ref=cacc_ref.at[1, pl.ds(lo, half)],
                send_sem=send_sems.at[h],
                recv_sem=recv_sems.at[h],
                device_id=peer,
                device_id_type=pl.DeviceIdType.MESH,
            )

        @pl.when(i == GRID // 2 - 1)
        def _():
            pl.semaphore_wait(barrier, 1)
            make_rdma(0).start()

        @pl.when(i == GRID - 1)
        def _():
            rdma1 = make_rdma(1)
            rdma1.start()
            rdma0 = make_rdma(0)
            rdma0.wait_send()
            rdma0.wait_recv()
            rdma1.wait_send()
            rdma1.wait_recv()

            out_ref[:, :] = (cacc_ref[0] + cacc_ref[1]) * (1.0 / (2 * n))

    out = pl.pallas_call(
        body,
        grid=(GRID,),
        out_shape=jax.ShapeDtypeStruct((m // 128, 128), jnp.float32),
        in_specs=[
            pl.BlockSpec((mc, n), lambda i: (i, 0), memory_space=pltpu.VMEM)
        ],
        out_specs=pl.BlockSpec(
            (m // 128, 128), lambda i: (0, 0), memory_space=pltpu.VMEM
        ),
        scratch_shapes=[
            pltpu.VMEM((2, m // 128, 128), jnp.float32),
            pltpu.SemaphoreType.DMA((2,)),
            pltpu.SemaphoreType.DMA((2,)),
        ],
        compiler_params=pltpu.CompilerParams(
            collective_id=0, dimension_semantics=("arbitrary",)
        ),
    )(x)
    return out.reshape(m, 1)


# device time: 11001 ns/iter; 3.1562x vs baseline; 1.0675x over previous
import jax
import jax.numpy as jnp
from jax import lax
from jax.experimental import pallas as pl
from jax.experimental.pallas import tpu as pltpu

GRID = 2


def kernel(x):
    m, n = x.shape
    mc = m // GRID
    r = mc // 128

    def body(x_ref, out_ref, cacc_ref, send_sems, recv_sems):
        i = pl.program_id(0)

        my_x = lax.axis_index("x")
        my_y = lax.axis_index("y")
        peer = (my_x, 1 - my_y)
        barrier = pltpu.get_barrier_semaphore()

        @pl.when(i == 0)
        def _():
            pl.semaphore_signal(
                barrier,
                inc=1,
                device_id=peer,
                device_id_type=pl.DeviceIdType.MESH,
            )

        parts = [x_ref[:, pl.ds(k * 128, 128)] for k in range(n // 128)]
        while len(parts) > 1:
            parts = [parts[j] + parts[j + 1] for j in range(0, len(parts), 2)]
        ones = jnp.ones((1, 128), jnp.float32)
        rs = lax.dot_general(
            ones,
            parts[0],
            dimension_numbers=(((1,), (1,)), ((), ())),
            preferred_element_type=jnp.float32,
        )
        for g in range(r):
            cacc_ref[0, pl.ds(i * r + g, 1), :] = rs[
                :, g * 128 : (g + 1) * 128
            ]

        half = (m // 128) // 2

        def make_rdma(h):
            lo = h * half
            return pltpu.make_async_remote_copy(
                src_ref=cacc_ref.at[0, pl.ds(lo, half)],
                dst_ref=cacc_ref.at[1, pl.ds(lo, half)],
                send_sem=send_sems.at[h],
                recv_sem=recv_sems.at[h],
                device_id=peer,
                device_id_type=pl.DeviceIdType.MESH,
            )

        @pl.when(i == GRID // 2 - 1)
        def _():
            pl.semaphore_wait(barrier, 1)
            make_rdma(0).start()

        @pl.when(i == GRID - 1)
        def _():
            rdma1 = make_rdma(1)
            rdma1.start()
            rdma0 = make_rdma(0)
            rdma0.wait_recv()
            rdma1.wait_recv()

            out_ref[:, :] = (cacc_ref[0] + cacc_ref[1]) * (1.0 / (2 * n))

            rdma0.wait_send()
            rdma1.wait_send()

    out = pl.pallas_call(
        body,
        grid=(GRID,),
        out_shape=jax.ShapeDtypeStruct((m // 128, 128), jnp.float32),
        in_specs=[
            pl.BlockSpec((mc, n), lambda i: (i, 0), memory_space=pltpu.VMEM)
        ],
        out_specs=pl.BlockSpec(
            (m // 128, 128), lambda i: (0, 0), memory_space=pltpu.VMEM
        ),
        scratch_shapes=[
            pltpu.VMEM((2, m // 128, 128), jnp.float32),
            pltpu.SemaphoreType.DMA((2,)),
            pltpu.SemaphoreType.DMA((2,)),
        ],
        compiler_params=pltpu.CompilerParams(
            collective_id=0, dimension_semantics=("arbitrary",)
        ),
    )(x)
    return out.reshape(m, 1)
